# baseline (speedup 1.0000x reference)
import functools

import jax
import jax.numpy as jnp
from jax.experimental import pallas as pl
from jax.experimental.pallas import tpu as pltpu

_ALPHA = 0.1
_B, _C, _W, _H = 16, 96, 224, 224
_N = _W * _H


def _blend_body(w_ref, x_ref, m_ref, o_ref):
    ch = pl.program_id(0)
    w = w_ref[ch]
    o_ref[...] = x_ref[...] + (m_ref[...] - x_ref[...]) * w


def kernel(x):
    B, C, W, H = x.shape
    kperm, klmda = jax.random.split(jax.random.key(42))
    perm = jax.random.permutation(kperm, B)
    lmda = jax.random.beta(klmda, _ALPHA, _ALPHA, (B,)).astype(x.dtype)

    src = x.reshape(B * C, W * H)
    order = jnp.argsort(src, axis=-1)
    sv = jnp.take_along_axis(src, order, axis=-1)
    n = W * H
    iota = jnp.broadcast_to(jnp.arange(n, dtype=jnp.int32), (B * C, n))
    ranks = jnp.zeros((B * C, n), jnp.int32).at[
        jnp.arange(B * C)[:, None], order
    ].set(iota)
    part = (perm[:, None] * C + jnp.arange(C)[None, :]).reshape(-1)
    matched = jnp.take_along_axis(sv[part], ranks, axis=-1)

    w = jnp.repeat(1.0 - lmda, C)
    src3 = src.reshape(B * C, n // 128, 128)
    matched3 = matched.reshape(B * C, n // 128, 128)
    out = pl.pallas_call(
        _blend_body,
        grid=(B * C,),
        in_specs=[
            pl.BlockSpec(memory_space=pltpu.SMEM),
            pl.BlockSpec((1, n // 128, 128), lambda i: (i, 0, 0)),
            pl.BlockSpec((1, n // 128, 128), lambda i: (i, 0, 0)),
        ],
        out_specs=pl.BlockSpec((1, n // 128, 128), lambda i: (i, 0, 0)),
        out_shape=jax.ShapeDtypeStruct((B * C, n // 128, 128), x.dtype),
    )(w, src3, matched3)
    return out.reshape(B, C, W, H)


# speedup vs baseline: 34.1915x; 34.1915x over previous
import functools

import jax
import jax.numpy as jnp
import numpy as np
from jax import lax
from jax.experimental import pallas as pl
from jax.experimental.pallas import tpu as pltpu
from jax.experimental.pallas import tpu_sc as plsc

_ALPHA = 0.1
_B, _C, _W, _H = 16, 96, 224, 224
_N = _W * _H
_NCH = _B * _C
_NBINS = 1 << 16
_CH = 6272
_NCHUNK = _N // _CH
_VPC = _CH // 16
_CPW = _C // 32

_PERM = tuple(
    int(v) for v in np.asarray(
        jax.random.permutation(jax.random.split(jax.random.key(42))[0], _B)
    )
)

_mesh = plsc.VectorSubcoreMesh(core_axis_name="c", subcore_axis_name="s")
_sc_params = pltpu.CompilerParams(needs_layout_passes=False)


def _key16(v):
    ub = plsc.bitcast(v, jnp.uint32)
    s = ub >> 31
    mask = (jnp.uint32(0) - s) | jnp.uint32(0x80000000)
    return ((ub ^ mask) >> 16).astype(jnp.int32)


@functools.partial(
    pl.kernel,
    out_type=(
        jax.ShapeDtypeStruct((_NCH * _N,), jnp.int32),
        jax.ShapeDtypeStruct((_NCH * _N,), jnp.float32),
    ),
    mesh=_mesh,
    compiler_params=_sc_params,
    scratch_types=[
        pltpu.VMEM((_NBINS,), jnp.int32),
        pltpu.VMEM((_N,), jnp.float32),
        pltpu.VMEM((_CH,), jnp.float32),
        pltpu.VMEM((_CH,), jnp.int32),
    ],
)
def _phase1(x_hbm, ranks_hbm, invcdf_hbm, cdf_v, inv_v, xb_v, rb_v):
    wid = lax.axis_index("c") * 16 + lax.axis_index("s")
    ones = jnp.ones((16,), jnp.int32)
    zeros = jnp.zeros((16,), jnp.int32)
    neginf = jnp.full((16,), -jnp.inf, jnp.float32)

    def do_channel(ch):
        base = ch * _N

        def zero_hist(i, _):
            cdf_v[pl.ds(i * 16, 16)] = zeros
            return 0
        lax.fori_loop(0, _NBINS // 16, zero_hist, 0)

        def hist_chunk(k, _):
            pltpu.sync_copy(x_hbm.at[pl.ds(base + k * _CH, _CH)], xb_v)

            def hist_vreg(j, _):
                b = _key16(xb_v[pl.ds(j * 16, 16)])
                plsc.addupdate_scatter(cdf_v, [b], ones)
                return 0
            lax.fori_loop(0, _VPC, hist_vreg, 0)
            return 0
        lax.fori_loop(0, _NCHUNK, hist_chunk, 0)

        def scan_vreg(i, carry):
            h = cdf_v[pl.ds(i * 16, 16)]
            s = plsc.cumsum(h)
            cdf_v[pl.ds(i * 16, 16)] = s - h + carry
            return carry + jnp.max(s)
        lax.fori_loop(0, _NBINS // 16, scan_vreg, jnp.int32(0))

        def init_inv(i, _):
            inv_v[pl.ds(i * 16, 16)] = neginf
            return 0
        lax.fori_loop(0, _N // 16, init_inv, 0)

        def rank_chunk(k, _):
            pltpu.sync_copy(x_hbm.at[pl.ds(base + k * _CH, _CH)], xb_v)

            def rank_vreg(j, _):
                v = xb_v[pl.ds(j * 16, 16)]
                b = _key16(v)
                r = plsc.load_gather(cdf_v, [b])
                rb_v[pl.ds(j * 16, 16)] = r
                plsc.store_scatter(inv_v, [r], v)
                return 0
            lax.fori_loop(0, _VPC, rank_vreg, 0)
            pltpu.sync_copy(rb_v, ranks_hbm.at[pl.ds(base + k * _CH, _CH)])
            return 0
        lax.fori_loop(0, _NCHUNK, rank_chunk, 0)

        def fill_vreg(i, carry):
            v = inv_v[pl.ds(i * 16, 16)]
            cm = jnp.maximum(plsc.cummax(v), carry)
            inv_v[pl.ds(i * 16, 16)] = cm
            return jnp.max(cm)
        lax.fori_loop(0, _N // 16, fill_vreg, jnp.float32(-jnp.inf))

        pltpu.sync_copy(inv_v, invcdf_hbm.at[pl.ds(base, _N)])

    def do_ci(ci, _):
        c = wid * _CPW + ci
        for b in range(_B):
            do_channel(b * _C + c)
        return 0
    lax.fori_loop(0, _CPW, do_ci, 0)


@functools.partial(
    pl.kernel,
    out_type=jax.ShapeDtypeStruct((_NCH * _N,), jnp.float32),
    mesh=_mesh,
    compiler_params=_sc_params,
    scratch_types=[
        pltpu.VMEM((_N,), jnp.float32),
        pltpu.VMEM((_CH,), jnp.int32),
        pltpu.VMEM((_CH,), jnp.float32),
        pltpu.VMEM((_CH,), jnp.float32),
        pltpu.VMEM((16,), jnp.float32),
    ],
)
def _phase2(x_hbm, ranks_hbm, invcdf_hbm, w_hbm, out_hbm,
            inv_v, rb_v, xb_v, ob_v, w_v):
    wid = lax.axis_index("c") * 16 + lax.axis_index("s")

    def do_channel(b, pb, c):
        ch = b * _C + c
        base = ch * _N
        pltpu.sync_copy(invcdf_hbm.at[pl.ds((pb * _C + c) * _N, _N)], inv_v)
        pltpu.sync_copy(w_hbm.at[pl.ds(b * 16, 16)], w_v)
        wv = w_v[...]

        def out_chunk(k, _):
            off = base + k * _CH
            pltpu.sync_copy(ranks_hbm.at[pl.ds(off, _CH)], rb_v)
            pltpu.sync_copy(x_hbm.at[pl.ds(off, _CH)], xb_v)

            def out_vreg(j, _):
                r = rb_v[pl.ds(j * 16, 16)]
                xv = xb_v[pl.ds(j * 16, 16)]
                m = plsc.load_gather(inv_v, [r])
                ob_v[pl.ds(j * 16, 16)] = xv + (m - xv) * wv
                return 0
            lax.fori_loop(0, _VPC, out_vreg, 0)
            pltpu.sync_copy(ob_v, out_hbm.at[pl.ds(off, _CH)])
            return 0
        lax.fori_loop(0, _NCHUNK, out_chunk, 0)

    def do_ci(ci, _):
        c = wid * _CPW + ci
        for b in range(_B):
            do_channel(b, _PERM[b], c)
        return 0
    lax.fori_loop(0, _CPW, do_ci, 0)


def kernel(x):
    B, C, W, H = x.shape
    _, klmda = jax.random.split(jax.random.key(42))
    lmda = jax.random.beta(klmda, _ALPHA, _ALPHA, (B,)).astype(x.dtype)
    w_rep = jnp.broadcast_to((1.0 - lmda)[:, None], (B, 16)).reshape(-1)

    xf = x.reshape(-1)
    ranks, invcdf = _phase1(xf)
    out = _phase2(xf, ranks, invcdf, w_rep)
    return out.reshape(B, C, W, H)


# speedup vs baseline: 70.5175x; 2.0624x over previous
import functools

import jax
import jax.numpy as jnp
import numpy as np
from jax import lax
from jax.experimental import pallas as pl
from jax.experimental.pallas import tpu as pltpu
from jax.experimental.pallas import tpu_sc as plsc

_ALPHA = 0.1
_B, _C, _W, _H = 16, 96, 224, 224
_N = _W * _H
_NCH = _B * _C
_NBINS = 1 << 16
_NV = _N // 16
_CH1 = 3136
_NC1 = _N // _CH1
_VC1 = _CH1 // 16
_CH2 = 6272
_NC2 = _N // _CH2
_VC2 = _CH2 // 16
_CPW = _C // 32

_PERM = tuple(
    int(v) for v in np.asarray(
        jax.random.permutation(jax.random.split(jax.random.key(42))[0], _B)
    )
)

_mesh = plsc.VectorSubcoreMesh(core_axis_name="c", subcore_axis_name="s")
_sc_params = pltpu.CompilerParams(needs_layout_passes=False)


def _keyaddr(v):
    ub = plsc.bitcast(v, jnp.uint32)
    s = ub >> 31
    u = ub ^ ((jnp.uint32(0) - s) | jnp.uint32(0x80000000))
    addr = ((u & jnp.uint32(0x0FFF0000)) >> 12) | (u >> 28)
    return addr.astype(jnp.int32)


@functools.partial(
    pl.kernel,
    out_type=(
        jax.ShapeDtypeStruct((_NCH * _N,), jnp.int32),
        jax.ShapeDtypeStruct((_NCH * _N,), jnp.float32),
    ),
    mesh=_mesh,
    compiler_params=_sc_params,
    scratch_types=[
        pltpu.VMEM((_NBINS,), jnp.int32),
        pltpu.VMEM((_N,), jnp.float32),
        pltpu.VMEM((_CH1,), jnp.float32),
        pltpu.VMEM((_CH1,), jnp.int32),
        pltpu.VMEM((_NV,), jnp.float32),
    ],
)
def _phase1(x_hbm, ranks_hbm, invcdf_hbm, cdf_v, inv_v, xb_v, rb_v, pf_v):
    wid = lax.axis_index("c") * 16 + lax.axis_index("s")
    ones = jnp.ones((16,), jnp.int32)
    zeros = jnp.zeros((16,), jnp.int32)
    neginf = jnp.full((16,), -jnp.inf, jnp.float32)
    vmaxidx = lax.iota(jnp.int32, 16) * 16 + 15

    def do_channel(ch):
        base = ch * _N

        def zero8(i, _):
            for j in range(8):
                cdf_v[pl.ds((i * 8 + j) * 16, 16)] = zeros
            return 0
        lax.fori_loop(0, _NBINS // 128, zero8, 0)

        def hist_chunk(k, _):
            pltpu.sync_copy(x_hbm.at[pl.ds(base + k * _CH1, _CH1)], xb_v)

            def hist4(i, _):
                for j in range(4):
                    a = _keyaddr(xb_v[pl.ds((i * 4 + j) * 16, 16)])
                    plsc.addupdate_scatter(cdf_v, [a], ones)
                return 0
            lax.fori_loop(0, _VC1 // 4, hist4, 0)
            return 0
        lax.fori_loop(0, _NC1, hist_chunk, 0)

        def sumA(i, acc):
            for j in range(8):
                acc = acc + cdf_v[pl.ds((i * 8 + j) * 16, 16)]
            return acc
        tot = lax.fori_loop(0, _NBINS // 128, sumA, zeros)
        run0 = plsc.cumsum(tot) - tot

        def scanC(i, run):
            for j in range(8):
                sl = pl.ds((i * 8 + j) * 16, 16)
                h = cdf_v[sl]
                cdf_v[sl] = run
                run = run + h
            return run
        lax.fori_loop(0, _NBINS // 128, scanC, run0)

        def init8(i, _):
            for j in range(8):
                inv_v[pl.ds((i * 8 + j) * 16, 16)] = neginf
            return 0
        lax.fori_loop(0, _NV // 8, init8, 0)

        def rank_chunk(k, _):
            pltpu.sync_copy(x_hbm.at[pl.ds(base + k * _CH1, _CH1)], xb_v)

            @plsc.parallel_loop(0, _VC1, 1, unroll=4)
            def rank_vreg(i):
                v = xb_v[pl.ds(i * 16, 16)]
                r = plsc.load_gather(cdf_v, [_keyaddr(v)])
                rb_v[pl.ds(i * 16, 16)] = r
                plsc.store_scatter(inv_v, [r], v)
            pltpu.sync_copy(rb_v, ranks_hbm.at[pl.ds(base + k * _CH1, _CH1)])
            return 0
        lax.fori_loop(0, _NC1, rank_chunk, 0)

        @plsc.parallel_loop(0, _NV, 1, unroll=8)
        def fillA(i):
            sl = pl.ds(i * 16, 16)
            inv_v[sl] = plsc.cummax(inv_v[sl])

        def fillB(g, carry):
            mx = plsc.load_gather(inv_v, [g * 256 + vmaxidx])
            pf = jnp.maximum(plsc.cummax(mx), carry)
            pf_v[pl.ds(g * 16, 16)] = pf
            return jnp.max(pf)
        lax.fori_loop(0, _NV // 16, fillB, jnp.float32(-jnp.inf))

        @plsc.parallel_loop(1, _NV, 1, unroll=8)
        def fillC(i):
            pv = plsc.load_gather(pf_v, [zeros + (i - 1)])
            sl = pl.ds(i * 16, 16)
            inv_v[sl] = jnp.maximum(inv_v[sl], pv)

        pltpu.sync_copy(inv_v, invcdf_hbm.at[pl.ds(base, _N)])

    def do_ci(ci, _):
        c = wid * _CPW + ci
        for b in range(_B):
            do_channel(b * _C + c)
        return 0
    lax.fori_loop(0, _CPW, do_ci, 0)


@functools.partial(
    pl.kernel,
    out_type=jax.ShapeDtypeStruct((_NCH * _N,), jnp.float32),
    mesh=_mesh,
    compiler_params=_sc_params,
    scratch_types=[
        pltpu.VMEM((_N,), jnp.float32),
        pltpu.VMEM((_CH2,), jnp.int32),
        pltpu.VMEM((_CH2,), jnp.float32),
        pltpu.VMEM((_CH2,), jnp.float32),
        pltpu.VMEM((16,), jnp.float32),
    ],
)
def _phase2(x_hbm, ranks_hbm, invcdf_hbm, w_hbm, out_hbm,
            inv_v, rb_v, xb_v, ob_v, w_v):
    wid = lax.axis_index("c") * 16 + lax.axis_index("s")

    def do_channel(b, pb, c):
        base = (b * _C + c) * _N
        pltpu.sync_copy(invcdf_hbm.at[pl.ds((pb * _C + c) * _N, _N)], inv_v)
        pltpu.sync_copy(w_hbm.at[pl.ds(b * 16, 16)], w_v)
        wv = w_v[...]

        def out_chunk(k, _):
            off = base + k * _CH2
            pltpu.sync_copy(ranks_hbm.at[pl.ds(off, _CH2)], rb_v)
            pltpu.sync_copy(x_hbm.at[pl.ds(off, _CH2)], xb_v)

            @plsc.parallel_loop(0, _VC2, 1, unroll=8)
            def out_vreg(i):
                sl = pl.ds(i * 16, 16)
                xv = xb_v[sl]
                m = plsc.load_gather(inv_v, [rb_v[sl]])
                ob_v[sl] = xv + (m - xv) * wv
            pltpu.sync_copy(ob_v, out_hbm.at[pl.ds(off, _CH2)])
            return 0
        lax.fori_loop(0, _NC2, out_chunk, 0)

    def do_ci(ci, _):
        c = wid * _CPW + ci
        for b in range(_B):
            do_channel(b, _PERM[b], c)
        return 0
    lax.fori_loop(0, _CPW, do_ci, 0)


def kernel(x):
    B, C, W, H = x.shape
    _, klmda = jax.random.split(jax.random.key(42))
    lmda = jax.random.beta(klmda, _ALPHA, _ALPHA, (B,)).astype(x.dtype)
    w_rep = jnp.broadcast_to((1.0 - lmda)[:, None], (B, 16)).reshape(-1)

    xf = x.reshape(-1)
    ranks, invcdf = _phase1(xf)
    out = _phase2(xf, ranks, invcdf, w_rep)
    return out.reshape(B, C, W, H)


# speedup vs baseline: 91.1950x; 1.2932x over previous
import functools

import jax
import jax.numpy as jnp
import numpy as np
from jax import lax
from jax.experimental import pallas as pl
from jax.experimental.pallas import tpu as pltpu
from jax.experimental.pallas import tpu_sc as plsc

_ALPHA = 0.1
_B, _C, _W, _H = 16, 96, 224, 224
_N = _W * _H
_NCH = _B * _C
_NBINS = 1 << 16
_NV = _N // 16
_CH = 3136
_NC = _N // _CH
_VC = _CH // 16
_CPW = _C // 32

_PERM = tuple(
    int(v) for v in np.asarray(
        jax.random.permutation(jax.random.split(jax.random.key(42))[0], _B)
    )
)

_mesh = plsc.VectorSubcoreMesh(core_axis_name="c", subcore_axis_name="s")
_sc_params = pltpu.CompilerParams(needs_layout_passes=False)


def _keyaddr(v):
    ub = plsc.bitcast(v, jnp.uint32)
    s = ub >> 31
    u = ub ^ ((jnp.uint32(0) - s) | jnp.uint32(0x80000000))
    addr = ((u >> 12) & jnp.uint32(0xFFF0)) | (u >> 28)
    return addr.astype(jnp.int32)


@functools.partial(
    pl.kernel,
    out_type=(
        jax.ShapeDtypeStruct((_NCH * _N,), jnp.int32),
        jax.ShapeDtypeStruct((_NCH * _N,), jnp.int32),
        jax.ShapeDtypeStruct((_NCH * _N,), jnp.float32),
    ),
    mesh=_mesh,
    compiler_params=_sc_params,
    scratch_types=[
        pltpu.VMEM((_NBINS,), jnp.int32),
        pltpu.VMEM((_N,), jnp.float32),
        pltpu.VMEM((_CH,), jnp.float32),
        pltpu.VMEM((_CH,), jnp.float32),
        pltpu.VMEM((_CH,), jnp.int32),
        pltpu.VMEM((_CH,), jnp.int32),
        pltpu.VMEM((16,), jnp.float32),
        pltpu.SemaphoreType.DMA,
        pltpu.SemaphoreType.DMA,
        pltpu.SemaphoreType.DMA,
        pltpu.SemaphoreType.DMA,
        pltpu.SemaphoreType.DMA,
        pltpu.SemaphoreType.DMA,
        pltpu.SemaphoreType.DMA,
        pltpu.SemaphoreType.DMA,
        pltpu.SemaphoreType.DMA,
        pltpu.SemaphoreType.DMA,
    ],
)
def _mix(x_hbm, w_hbm, out_hbm, ranks_hbm, invcdf_hbm,
         cdf_v, inv_v, xb0, xb1, rb0, rb1, wb_v,
         sx0, sx1, sr0, sr1, sw0, sw1, so0, so1, sv, si):
    wid = lax.axis_index("c") * 16 + lax.axis_index("s")
    ones = jnp.ones((16,), jnp.int32)
    zeros = jnp.zeros((16,), jnp.int32)
    neginf = jnp.full((16,), -jnp.inf, jnp.float32)
    vmaxidx = lax.iota(jnp.int32, 16) * 16 + 15

    def wait_read(buf, sem):
        pltpu.make_async_copy(x_hbm.at[pl.ds(0, _CH)], buf, sem).wait()

    def wait_write(buf, hbm, sem):
        pltpu.make_async_copy(buf, hbm.at[pl.ds(0, _CH)], sem).wait()

    def phase1(b, c):
        ch = b * _C + c
        base = ch * _N

        def zero8(i, _):
            for j in range(8):
                cdf_v[pl.ds((i * 8 + j) * 16, 16)] = zeros
            return 0
        lax.fori_loop(0, _NBINS // 128, zero8, 0)

        def hist_buf(xb):
            def hist4(i, _):
                for j in range(4):
                    a = _keyaddr(xb[pl.ds((i * 4 + j) * 16, 16)])
                    plsc.addupdate_scatter(cdf_v, [a], ones)
                return 0
            lax.fori_loop(0, _VC // 4, hist4, 0)

        pltpu.async_copy(x_hbm.at[pl.ds(base, _CH)], xb0, sx0)

        def histg(g, _):
            k0 = 2 * g
            pltpu.async_copy(x_hbm.at[pl.ds(base + (k0 + 1) * _CH, _CH)],
                             xb1, sx1)
            wait_read(xb0, sx0)
            hist_buf(xb0)

            @pl.when(k0 + 2 < _NC)
            def _():
                pltpu.async_copy(x_hbm.at[pl.ds(base + (k0 + 2) * _CH, _CH)],
                                 xb0, sx0)
            wait_read(xb1, sx1)
            hist_buf(xb1)
            return 0
        lax.fori_loop(0, _NC // 2, histg, 0)

        def sumA(i, acc):
            for j in range(8):
                acc = acc + cdf_v[pl.ds((i * 8 + j) * 16, 16)]
            return acc
        tot = lax.fori_loop(0, _NBINS // 128, sumA, zeros)
        run0 = plsc.cumsum(tot) - tot

        def scanC(i, run):
            for j in range(8):
                sl = pl.ds((i * 8 + j) * 16, 16)
                h = cdf_v[sl]
                cdf_v[sl] = run
                run = run + h
            return run
        lax.fori_loop(0, _NBINS // 128, scanC, run0)

        @pl.when(b > 0)
        def _():
            pltpu.make_async_copy(inv_v, invcdf_hbm.at[pl.ds(0, _N)],
                                  sv).wait()

        def init8(i, _):
            for j in range(8):
                inv_v[pl.ds((i * 8 + j) * 16, 16)] = neginf
            return 0
        lax.fori_loop(0, _NV // 8, init8, 0)

        def rank_buf(xb, rb):
            @plsc.parallel_loop(0, _VC, 1, unroll=4)
            def rank_vreg(i):
                v = xb[pl.ds(i * 16, 16)]
                r = plsc.load_gather(cdf_v, [_keyaddr(v)])
                rb[pl.ds(i * 16, 16)] = r
                plsc.store_scatter(inv_v, [r], v)

        pltpu.async_copy(x_hbm.at[pl.ds(base, _CH)], xb0, sx0)

        def rankg(g, _):
            k0 = 2 * g
            pltpu.async_copy(x_hbm.at[pl.ds(base + (k0 + 1) * _CH, _CH)],
                             xb1, sx1)
            wait_read(xb0, sx0)

            @pl.when(g > 0)
            def _():
                wait_write(rb0, ranks_hbm, sw0)
            rank_buf(xb0, rb0)
            pltpu.async_copy(rb0, ranks_hbm.at[pl.ds(base + k0 * _CH, _CH)],
                             sw0)

            @pl.when(k0 + 2 < _NC)
            def _():
                pltpu.async_copy(x_hbm.at[pl.ds(base + (k0 + 2) * _CH, _CH)],
                                 xb0, sx0)
            wait_read(xb1, sx1)

            @pl.when(g > 0)
            def _():
                wait_write(rb1, ranks_hbm, sw1)
            rank_buf(xb1, rb1)
            pltpu.async_copy(rb1,
                             ranks_hbm.at[pl.ds(base + (k0 + 1) * _CH, _CH)],
                             sw1)
            return 0
        lax.fori_loop(0, _NC // 2, rankg, 0)
        wait_write(rb0, ranks_hbm, sw0)
        wait_write(rb1, ranks_hbm, sw1)

        @plsc.parallel_loop(0, _NV, 1, unroll=8)
        def fillA(i):
            sl = pl.ds(i * 16, 16)
            inv_v[sl] = plsc.cummax(inv_v[sl])

        def fillB(g, carry):
            mx = plsc.load_gather(inv_v, [g * 256 + vmaxidx])
            pf = jnp.maximum(plsc.cummax(mx), carry)
            xb0[pl.ds(g * 16, 16)] = pf
            return jnp.max(pf)
        lax.fori_loop(0, _NV // 16, fillB, jnp.float32(-jnp.inf))

        @plsc.parallel_loop(1, _NV, 1, unroll=8)
        def fillC(i):
            pv = plsc.load_gather(xb0, [zeros + (i - 1)])
            sl = pl.ds(i * 16, 16)
            inv_v[sl] = jnp.maximum(inv_v[sl], pv)

        pltpu.async_copy(inv_v, invcdf_hbm.at[pl.ds(base, _N)], sv)

    def phase2(b, pb, c):
        base = (b * _C + c) * _N
        if b == 0:
            pltpu.make_async_copy(inv_v, invcdf_hbm.at[pl.ds(0, _N)],
                                  sv).wait()
        pltpu.async_copy(invcdf_hbm.at[pl.ds((pb * _C + c) * _N, _N)],
                         inv_v, si).wait()
        pltpu.async_copy(w_hbm.at[pl.ds(b * 16, 16)], wb_v, si).wait()
        wv = wb_v[...]

        def out_buf(xb, rb, slot):
            @plsc.parallel_loop(0, _VC, 1, unroll=8)
            def out_vreg(i):
                sl = pl.ds(i * 16, 16)
                xv = xb[sl]
                m = plsc.load_gather(inv_v, [rb[sl]])
                cdf_v[pl.ds(slot * _CH + i * 16, 16)] = plsc.bitcast(
                    xv + (m - xv) * wv, jnp.int32)

        pltpu.async_copy(ranks_hbm.at[pl.ds(base, _CH)], rb0, sr0)
        pltpu.async_copy(x_hbm.at[pl.ds(base, _CH)], xb0, sx0)

        def outg(g, _):
            k0 = 2 * g
            pltpu.async_copy(ranks_hbm.at[pl.ds(base + (k0 + 1) * _CH, _CH)],
                             rb1, sr1)
            pltpu.async_copy(x_hbm.at[pl.ds(base + (k0 + 1) * _CH, _CH)],
                             xb1, sx1)
            wait_read(xb0, sx0)
            wait_read(rb0, sr0)

            @pl.when(g > 0)
            def _():
                wait_write(rb0, out_hbm, so0)
            out_buf(xb0, rb0, 0)
            pltpu.async_copy(cdf_v.at[pl.ds(0, _CH)],
                             out_hbm.at[pl.ds(base + k0 * _CH, _CH)], so0)

            @pl.when(k0 + 2 < _NC)
            def _():
                pltpu.async_copy(
                    ranks_hbm.at[pl.ds(base + (k0 + 2) * _CH, _CH)], rb0, sr0)
                pltpu.async_copy(
                    x_hbm.at[pl.ds(base + (k0 + 2) * _CH, _CH)], xb0, sx0)
            wait_read(xb1, sx1)
            wait_read(rb1, sr1)

            @pl.when(g > 0)
            def _():
                wait_write(rb1, out_hbm, so1)
            out_buf(xb1, rb1, 1)
            pltpu.async_copy(cdf_v.at[pl.ds(_CH, _CH)],
                             out_hbm.at[pl.ds(base + (k0 + 1) * _CH, _CH)],
                             so1)
            return 0
        lax.fori_loop(0, _NC // 2, outg, 0)
        wait_write(rb0, out_hbm, so0)
        wait_write(rb1, out_hbm, so1)

    def do_ci(ci, _):
        c = wid * _CPW + ci

        def do_b(b, _):
            phase1(b, c)
            return 0
        lax.fori_loop(0, _B, do_b, 0)
        for b in range(_B):
            phase2(b, _PERM[b], c)
        return 0
    lax.fori_loop(0, _CPW, do_ci, 0)


def kernel(x):
    B, C, W, H = x.shape
    _, klmda = jax.random.split(jax.random.key(42))
    lmda = jax.random.beta(klmda, _ALPHA, _ALPHA, (B,)).astype(x.dtype)
    w_rep = jnp.broadcast_to((1.0 - lmda)[:, None], (B, 16)).reshape(-1)

    out_i32, _, _ = _mix(x.reshape(-1), w_rep)
    return lax.bitcast_convert_type(out_i32, jnp.float32).reshape(B, C, W, H)


# speedup vs baseline: 94.2736x; 1.0338x over previous
import functools

import jax
import jax.numpy as jnp
import numpy as np
from jax import lax
from jax.experimental import pallas as pl
from jax.experimental.pallas import tpu as pltpu
from jax.experimental.pallas import tpu_sc as plsc

_ALPHA = 0.1
_B, _C, _W, _H = 16, 96, 224, 224
_N = _W * _H
_NCH = _B * _C
_NBINS = 1 << 16
_NV = _N // 16
_CH = 3136
_NC = _N // _CH
_VC = _CH // 16
_CPW = _C // 32

_PERM = (1, 3, 9, 11, 5, 15, 0, 14, 2, 12, 6, 7, 13, 10, 4, 8)

_mesh = plsc.VectorSubcoreMesh(core_axis_name="c", subcore_axis_name="s")
_sc_params = pltpu.CompilerParams(needs_layout_passes=False)


def _keyaddr(v):
    bi = plsc.bitcast(v, jnp.int32)
    t = bi ^ (bi >> 31)
    lane = (plsc.bitcast(v, jnp.uint32) >> 28).astype(jnp.int32)
    return ((t >> 12) & jnp.int32(0xFFF0)) | lane


@functools.partial(
    pl.kernel,
    out_type=(
        jax.ShapeDtypeStruct((_NCH * _N,), jnp.int32),
        jax.ShapeDtypeStruct((_NCH * _N,), jnp.int32),
        jax.ShapeDtypeStruct((_NCH * _N,), jnp.float32),
    ),
    mesh=_mesh,
    compiler_params=_sc_params,
    scratch_types=[
        pltpu.VMEM((_NBINS,), jnp.int32),
        pltpu.VMEM((_N,), jnp.float32),
        pltpu.VMEM((_CH,), jnp.float32),
        pltpu.VMEM((_CH,), jnp.float32),
        pltpu.VMEM((_CH,), jnp.int32),
        pltpu.VMEM((_CH,), jnp.int32),
        pltpu.VMEM((16,), jnp.float32),
        pltpu.SemaphoreType.DMA,
        pltpu.SemaphoreType.DMA,
        pltpu.SemaphoreType.DMA,
        pltpu.SemaphoreType.DMA,
        pltpu.SemaphoreType.DMA,
        pltpu.SemaphoreType.DMA,
        pltpu.SemaphoreType.DMA,
        pltpu.SemaphoreType.DMA,
        pltpu.SemaphoreType.DMA,
        pltpu.SemaphoreType.DMA,
    ],
)
def _mix(x_hbm, w_hbm, out_hbm, ranks_hbm, invcdf_hbm,
         cdf_v, inv_v, xb0, xb1, rb0, rb1, wb_v,
         sx0, sx1, sr0, sr1, sw0, sw1, so0, so1, sv, si):
    wid = lax.axis_index("c") * 16 + lax.axis_index("s")
    ones = jnp.ones((16,), jnp.int32)
    zeros = jnp.zeros((16,), jnp.int32)
    neginf = jnp.full((16,), -jnp.inf, jnp.float32)
    vmaxidx = lax.iota(jnp.int32, 16) * 16 + 15

    def wait_read(buf, sem):
        pltpu.make_async_copy(x_hbm.at[pl.ds(0, _CH)], buf, sem).wait()

    def wait_write(buf, hbm, sem):
        pltpu.make_async_copy(buf, hbm.at[pl.ds(0, _CH)], sem).wait()

    def phase1(b, c):
        ch = b * _C + c
        base = ch * _N

        @pl.when(b == 0)
        def _():
            def zero8(i, _):
                for j in range(8):
                    cdf_v[pl.ds((i * 8 + j) * 16, 16)] = zeros
                return 0
            lax.fori_loop(0, _NBINS // 128, zero8, 0)

        @pl.when(b > 0)
        def _():
            pltpu.make_async_copy(inv_v, invcdf_hbm.at[pl.ds(0, _N)],
                                  sv).wait()

        def hist_buf(xb, k):
            def hist4(i, _):
                for j in range(4):
                    a = _keyaddr(xb[pl.ds((i * 4 + j) * 16, 16)])
                    plsc.addupdate_scatter(cdf_v, [a], ones)
                    inv_v[pl.ds((k * _VC + i * 4 + j) * 16, 16)] = neginf
                return 0
            lax.fori_loop(0, _VC // 4, hist4, 0)

        pltpu.async_copy(x_hbm.at[pl.ds(base, _CH)], xb0, sx0)

        def histg(g, _):
            k0 = 2 * g
            pltpu.async_copy(x_hbm.at[pl.ds(base + (k0 + 1) * _CH, _CH)],
                             xb1, sx1)
            wait_read(xb0, sx0)
            hist_buf(xb0, k0)

            @pl.when(k0 + 2 < _NC)
            def _():
                pltpu.async_copy(x_hbm.at[pl.ds(base + (k0 + 2) * _CH, _CH)],
                                 xb0, sx0)
            wait_read(xb1, sx1)
            hist_buf(xb1, k0 + 1)
            return 0
        lax.fori_loop(0, _NC // 2, histg, 0)

        def sumA(i, acc):
            for j in range(8):
                acc = acc + cdf_v[pl.ds((i * 8 + j) * 16, 16)]
            return acc
        tot = lax.fori_loop(0, _NBINS // 128, sumA, zeros)
        ct = plsc.cumsum(tot)
        ct15 = jnp.max(ct)
        poslane = lax.iota(jnp.int32, 16) < 8
        possum = jnp.sum(jnp.where(poslane, tot, zeros))
        run0 = jnp.where(poslane, (ct15 - possum) + ct - tot, ct15 - ct)

        def scanC(i, run):
            for j in range(8):
                sl = pl.ds((i * 8 + j) * 16, 16)
                h = cdf_v[sl]
                cdf_v[sl] = run
                run = run + h
            return run
        lax.fori_loop(0, _NBINS // 128, scanC, run0)

        def rank_buf(xb, rb):
            @plsc.parallel_loop(0, _VC, 1, unroll=7)
            def rank_vreg(i):
                v = xb[pl.ds(i * 16, 16)]
                r = plsc.load_gather(cdf_v, [_keyaddr(v)])
                rb[pl.ds(i * 16, 16)] = r
                plsc.store_scatter(inv_v, [r], v)

        pltpu.async_copy(x_hbm.at[pl.ds(base, _CH)], xb0, sx0)

        def rankg(g, _):
            k0 = 2 * g
            pltpu.async_copy(x_hbm.at[pl.ds(base + (k0 + 1) * _CH, _CH)],
                             xb1, sx1)
            wait_read(xb0, sx0)

            @pl.when(g > 0)
            def _():
                wait_write(rb0, ranks_hbm, sw0)
            rank_buf(xb0, rb0)
            pltpu.async_copy(rb0, ranks_hbm.at[pl.ds(base + k0 * _CH, _CH)],
                             sw0)

            @pl.when(k0 + 2 < _NC)
            def _():
                pltpu.async_copy(x_hbm.at[pl.ds(base + (k0 + 2) * _CH, _CH)],
                                 xb0, sx0)
            wait_read(xb1, sx1)

            @pl.when(g > 0)
            def _():
                wait_write(rb1, ranks_hbm, sw1)
            rank_buf(xb1, rb1)
            pltpu.async_copy(rb1,
                             ranks_hbm.at[pl.ds(base + (k0 + 1) * _CH, _CH)],
                             sw1)
            return 0
        lax.fori_loop(0, _NC // 2, rankg, 0)
        wait_write(rb0, ranks_hbm, sw0)
        wait_write(rb1, ranks_hbm, sw1)

        @plsc.parallel_loop(0, _NV, 1, unroll=8)
        def fillA(i):
            sl = pl.ds(i * 16, 16)
            inv_v[sl] = plsc.cummax(inv_v[sl])
            cdf_v[sl] = zeros

        @plsc.parallel_loop(_NV, _NBINS // 16, 1, unroll=8)
        def zrest(i):
            cdf_v[pl.ds(i * 16, 16)] = zeros

        def fillB(g, carry):
            mx = plsc.load_gather(inv_v, [g * 256 + vmaxidx])
            pf = jnp.maximum(plsc.cummax(mx), carry)
            xb0[pl.ds(g * 16, 16)] = pf
            return jnp.max(pf)
        lax.fori_loop(0, _NV // 16, fillB, jnp.float32(-jnp.inf))

        @plsc.parallel_loop(1, _NV, 1, unroll=8)
        def fillC(i):
            pv = plsc.load_gather(xb0, [zeros + (i - 1)])
            sl = pl.ds(i * 16, 16)
            inv_v[sl] = jnp.maximum(inv_v[sl], pv)

        pltpu.async_copy(inv_v, invcdf_hbm.at[pl.ds(base, _N)], sv)

    def phase2(b, pb, c):
        base = (b * _C + c) * _N
        if b == 0:
            pltpu.make_async_copy(inv_v, invcdf_hbm.at[pl.ds(0, _N)],
                                  sv).wait()
        pltpu.async_copy(invcdf_hbm.at[pl.ds((pb * _C + c) * _N, _N)],
                         inv_v, si).wait()
        pltpu.async_copy(w_hbm.at[pl.ds(b * 16, 16)], wb_v, si).wait()
        wv = wb_v[...]

        def out_buf(xb, rb, slot):
            @plsc.parallel_loop(0, _VC, 1, unroll=7)
            def out_vreg(i):
                sl = pl.ds(i * 16, 16)
                xv = xb[sl]
                m = plsc.load_gather(inv_v, [rb[sl]])
                cdf_v[pl.ds(slot * _CH + i * 16, 16)] = plsc.bitcast(
                    xv + (m - xv) * wv, jnp.int32)

        pltpu.async_copy(ranks_hbm.at[pl.ds(base, _CH)], rb0, sr0)
        pltpu.async_copy(x_hbm.at[pl.ds(base, _CH)], xb0, sx0)

        def outg(g, _):
            k0 = 2 * g
            pltpu.async_copy(ranks_hbm.at[pl.ds(base + (k0 + 1) * _CH, _CH)],
                             rb1, sr1)
            pltpu.async_copy(x_hbm.at[pl.ds(base + (k0 + 1) * _CH, _CH)],
                             xb1, sx1)
            wait_read(xb0, sx0)
            wait_read(rb0, sr0)

            @pl.when(g > 0)
            def _():
                wait_write(rb0, out_hbm, so0)
            out_buf(xb0, rb0, 0)
            pltpu.async_copy(cdf_v.at[pl.ds(0, _CH)],
                             out_hbm.at[pl.ds(base + k0 * _CH, _CH)], so0)

            @pl.when(k0 + 2 < _NC)
            def _():
                pltpu.async_copy(
                    ranks_hbm.at[pl.ds(base + (k0 + 2) * _CH, _CH)], rb0, sr0)
                pltpu.async_copy(
                    x_hbm.at[pl.ds(base + (k0 + 2) * _CH, _CH)], xb0, sx0)
            wait_read(xb1, sx1)
            wait_read(rb1, sr1)

            @pl.when(g > 0)
            def _():
                wait_write(rb1, out_hbm, so1)
            out_buf(xb1, rb1, 1)
            pltpu.async_copy(cdf_v.at[pl.ds(_CH, _CH)],
                             out_hbm.at[pl.ds(base + (k0 + 1) * _CH, _CH)],
                             so1)
            return 0
        lax.fori_loop(0, _NC // 2, outg, 0)
        wait_write(rb0, out_hbm, so0)
        wait_write(rb1, out_hbm, so1)

    def do_ci(ci, _):
        c = wid * _CPW + ci

        def do_b(b, _):
            phase1(b, c)
            return 0
        lax.fori_loop(0, _B, do_b, 0)
        for b in range(_B):
            phase2(b, _PERM[b], c)
        return 0
    lax.fori_loop(0, _CPW, do_ci, 0)


def kernel(x):
    B, C, W, H = x.shape
    _, klmda = jax.random.split(jax.random.key(42))
    lmda = jax.random.beta(klmda, _ALPHA, _ALPHA, (B,)).astype(x.dtype)
    w_rep = jnp.broadcast_to((1.0 - lmda)[:, None], (B, 16)).reshape(-1)

    out_i32, _, _ = _mix(x.reshape(-1), w_rep)
    return lax.bitcast_convert_type(out_i32, jnp.float32).reshape(B, C, W, H)


# speedup vs baseline: 95.5818x; 1.0139x over previous
import functools

import jax
import jax.numpy as jnp
import numpy as np
from jax import lax
from jax.experimental import pallas as pl
from jax.experimental.pallas import tpu as pltpu
from jax.experimental.pallas import tpu_sc as plsc

_ALPHA = 0.1
_B, _C, _W, _H = 16, 96, 224, 224
_N = _W * _H
_NCH = _B * _C
_NBINS = 1 << 16
_NV = _N // 16
_CH = 3136
_NC = _N // _CH
_VC = _CH // 16
_CPW = _C // 32

_PERM = (1, 3, 9, 11, 5, 15, 0, 14, 2, 12, 6, 7, 13, 10, 4, 8)

_mesh = plsc.VectorSubcoreMesh(core_axis_name="c", subcore_axis_name="s")
_sc_params = pltpu.CompilerParams(needs_layout_passes=False)


def _keyaddr(v):
    bi = plsc.bitcast(v, jnp.int32)
    t = bi ^ (bi >> 31)
    lane = (plsc.bitcast(v, jnp.uint32) >> 28).astype(jnp.int32)
    return ((t >> 12) & jnp.int32(0xFFF0)) | lane


@functools.partial(
    pl.kernel,
    out_type=(
        jax.ShapeDtypeStruct((_NCH * _N,), jnp.int32),
        jax.ShapeDtypeStruct((_NCH * _N,), jnp.int32),
        jax.ShapeDtypeStruct((_NCH * _N,), jnp.float32),
    ),
    mesh=_mesh,
    compiler_params=_sc_params,
    scratch_types=[
        pltpu.VMEM((_NBINS,), jnp.int32),
        pltpu.VMEM((_N,), jnp.float32),
        pltpu.VMEM((_CH,), jnp.float32),
        pltpu.VMEM((_CH,), jnp.float32),
        pltpu.VMEM((_CH,), jnp.int32),
        pltpu.VMEM((_CH,), jnp.int32),
        pltpu.VMEM((16,), jnp.float32),
        pltpu.SemaphoreType.DMA,
        pltpu.SemaphoreType.DMA,
        pltpu.SemaphoreType.DMA,
        pltpu.SemaphoreType.DMA,
        pltpu.SemaphoreType.DMA,
        pltpu.SemaphoreType.DMA,
        pltpu.SemaphoreType.DMA,
        pltpu.SemaphoreType.DMA,
        pltpu.SemaphoreType.DMA,
        pltpu.SemaphoreType.DMA,
        pltpu.SemaphoreType.DMA,
        pltpu.SemaphoreType.DMA,
        pltpu.VMEM_SHARED((_NBINS // 8,), jnp.int32),
        pltpu.VMEM_SHARED((_N // 2,), jnp.float32),
    ],
)
def _mix(x_hbm, w_hbm, out_hbm, ranks_hbm, invcdf_hbm,
         cdf_v, inv_v, xb0, xb1, rb0, rb1, wb_v,
         sx0, sx1, sr0, sr1, sw0, sw1, so0, so1, sv, si, sz, sn,
         zero_sp, neg_sp):
    wid = lax.axis_index("c") * 16 + lax.axis_index("s")
    sid = lax.axis_index("s")
    ones = jnp.ones((16,), jnp.int32)
    zeros = jnp.zeros((16,), jnp.int32)
    neginf = jnp.full((16,), -jnp.inf, jnp.float32)
    vmaxidx = lax.iota(jnp.int32, 16) * 16 + 15

    def cfill(i, _):
        for j in range(4):
            rb0[pl.ds((i * 4 + j) * 16, 16)] = zeros
            xb0[pl.ds((i * 4 + j) * 16, 16)] = neginf
        return 0
    lax.fori_loop(0, _VC // 4, cfill, 0)
    pltpu.sync_copy(rb0.at[pl.ds(0, 512)], zero_sp.at[pl.ds(sid * 512, 512)])
    pltpu.sync_copy(xb0.at[pl.ds(0, _N // 32)],
                    neg_sp.at[pl.ds(sid * (_N // 32), _N // 32)])
    plsc.subcore_barrier()

    def wait_read(buf, sem):
        pltpu.make_async_copy(x_hbm.at[pl.ds(0, _CH)], buf, sem).wait()

    def wait_write(buf, hbm, sem):
        pltpu.make_async_copy(buf, hbm.at[pl.ds(0, _CH)], sem).wait()

    def phase1(b, c):
        ch = b * _C + c
        base = ch * _N

        @pl.when(b == 0)
        def _():
            for q in range(8):
                pltpu.sync_copy(zero_sp,
                                cdf_v.at[pl.ds(q * (_NBINS // 8), _NBINS // 8)])

        @pl.when(b > 0)
        def _():
            for q in range(8):
                pltpu.make_async_copy(
                    zero_sp, cdf_v.at[pl.ds(0, _NBINS // 8)], sz).wait()
            pltpu.make_async_copy(inv_v, invcdf_hbm.at[pl.ds(0, _N)],
                                  sv).wait()
        pltpu.async_copy(neg_sp, inv_v.at[pl.ds(0, _N // 2)], sn)
        pltpu.async_copy(neg_sp, inv_v.at[pl.ds(_N // 2, _N // 2)], sn)

        def hist_buf(xb):
            def hist4(i, _):
                for j in range(4):
                    a = _keyaddr(xb[pl.ds((i * 4 + j) * 16, 16)])
                    plsc.addupdate_scatter(cdf_v, [a], ones)
                return 0
            lax.fori_loop(0, _VC // 4, hist4, 0)

        pltpu.async_copy(x_hbm.at[pl.ds(base, _CH)], xb0, sx0)

        def histg(g, _):
            k0 = 2 * g
            pltpu.async_copy(x_hbm.at[pl.ds(base + (k0 + 1) * _CH, _CH)],
                             xb1, sx1)
            wait_read(xb0, sx0)
            hist_buf(xb0)

            @pl.when(k0 + 2 < _NC)
            def _():
                pltpu.async_copy(x_hbm.at[pl.ds(base + (k0 + 2) * _CH, _CH)],
                                 xb0, sx0)
            wait_read(xb1, sx1)
            hist_buf(xb1)
            return 0
        lax.fori_loop(0, _NC // 2, histg, 0)

        def sumA(i, acc):
            for j in range(8):
                acc = acc + cdf_v[pl.ds((i * 8 + j) * 16, 16)]
            return acc
        tot = lax.fori_loop(0, _NBINS // 128, sumA, zeros)
        ct = plsc.cumsum(tot)
        ct15 = jnp.max(ct)
        poslane = lax.iota(jnp.int32, 16) < 8
        possum = jnp.sum(jnp.where(poslane, tot, zeros))
        run0 = jnp.where(poslane, (ct15 - possum) + ct - tot, ct15 - ct)

        def scanC(i, run):
            for j in range(8):
                sl = pl.ds((i * 8 + j) * 16, 16)
                h = cdf_v[sl]
                cdf_v[sl] = run
                run = run + h
            return run
        lax.fori_loop(0, _NBINS // 128, scanC, run0)

        pltpu.make_async_copy(neg_sp, inv_v.at[pl.ds(0, _N // 2)], sn).wait()
        pltpu.make_async_copy(neg_sp, inv_v.at[pl.ds(0, _N // 2)], sn).wait()

        def rank_buf(xb, rb):
            @plsc.parallel_loop(0, _VC, 1, unroll=7)
            def rank_vreg(i):
                v = xb[pl.ds(i * 16, 16)]
                r = plsc.load_gather(cdf_v, [_keyaddr(v)])
                rb[pl.ds(i * 16, 16)] = r
                plsc.store_scatter(inv_v, [r], v)

        pltpu.async_copy(x_hbm.at[pl.ds(base, _CH)], xb0, sx0)

        def rankg(g, _):
            k0 = 2 * g
            pltpu.async_copy(x_hbm.at[pl.ds(base + (k0 + 1) * _CH, _CH)],
                             xb1, sx1)
            wait_read(xb0, sx0)

            @pl.when(g > 0)
            def _():
                wait_write(rb0, ranks_hbm, sw0)
            rank_buf(xb0, rb0)
            pltpu.async_copy(rb0, ranks_hbm.at[pl.ds(base + k0 * _CH, _CH)],
                             sw0)

            @pl.when(k0 + 2 < _NC)
            def _():
                pltpu.async_copy(x_hbm.at[pl.ds(base + (k0 + 2) * _CH, _CH)],
                                 xb0, sx0)
            wait_read(xb1, sx1)

            @pl.when(g > 0)
            def _():
                wait_write(rb1, ranks_hbm, sw1)
            rank_buf(xb1, rb1)
            pltpu.async_copy(rb1,
                             ranks_hbm.at[pl.ds(base + (k0 + 1) * _CH, _CH)],
                             sw1)
            return 0
        lax.fori_loop(0, _NC // 2, rankg, 0)
        wait_write(rb0, ranks_hbm, sw0)
        wait_write(rb1, ranks_hbm, sw1)

        @pl.when(b < _B - 1)
        def _():
            for q in range(8):
                pltpu.async_copy(
                    zero_sp, cdf_v.at[pl.ds(q * (_NBINS // 8), _NBINS // 8)],
                    sz)

        @plsc.parallel_loop(0, _NV, 1, unroll=8)
        def fillA(i):
            sl = pl.ds(i * 16, 16)
            inv_v[sl] = plsc.cummax(inv_v[sl])

        def fillB(g, carry):
            mx = plsc.load_gather(inv_v, [g * 256 + vmaxidx])
            pf = jnp.maximum(plsc.cummax(mx), carry)
            xb0[pl.ds(g * 16, 16)] = pf
            return jnp.max(pf)
        lax.fori_loop(0, _NV // 16, fillB, jnp.float32(-jnp.inf))

        @plsc.parallel_loop(1, _NV, 1, unroll=8)
        def fillC(i):
            pv = plsc.load_gather(xb0, [zeros + (i - 1)])
            sl = pl.ds(i * 16, 16)
            inv_v[sl] = jnp.maximum(inv_v[sl], pv)

        pltpu.async_copy(inv_v, invcdf_hbm.at[pl.ds(base, _N)], sv)

    def phase2(b, pb, c):
        base = (b * _C + c) * _N
        if b == 0:
            pltpu.make_async_copy(inv_v, invcdf_hbm.at[pl.ds(0, _N)],
                                  sv).wait()
        pltpu.async_copy(invcdf_hbm.at[pl.ds((pb * _C + c) * _N, _N)],
                         inv_v, si).wait()
        pltpu.async_copy(w_hbm.at[pl.ds(b * 16, 16)], wb_v, si).wait()
        wv = wb_v[...]

        def out_buf(xb, rb, slot):
            @plsc.parallel_loop(0, _VC, 1, unroll=7)
            def out_vreg(i):
                sl = pl.ds(i * 16, 16)
                xv = xb[sl]
                m = plsc.load_gather(inv_v, [rb[sl]])
                cdf_v[pl.ds(slot * _CH + i * 16, 16)] = plsc.bitcast(
                    xv + (m - xv) * wv, jnp.int32)

        pltpu.async_copy(ranks_hbm.at[pl.ds(base, _CH)], rb0, sr0)
        pltpu.async_copy(x_hbm.at[pl.ds(base, _CH)], xb0, sx0)

        def outg(g, _):
            k0 = 2 * g
            pltpu.async_copy(ranks_hbm.at[pl.ds(base + (k0 + 1) * _CH, _CH)],
                             rb1, sr1)
            pltpu.async_copy(x_hbm.at[pl.ds(base + (k0 + 1) * _CH, _CH)],
                             xb1, sx1)
            wait_read(xb0, sx0)
            wait_read(rb0, sr0)

            @pl.when(g > 0)
            def _():
                wait_write(rb0, out_hbm, so0)
            out_buf(xb0, rb0, 0)
            pltpu.async_copy(cdf_v.at[pl.ds(0, _CH)],
                             out_hbm.at[pl.ds(base + k0 * _CH, _CH)], so0)

            @pl.when(k0 + 2 < _NC)
            def _():
                pltpu.async_copy(
                    ranks_hbm.at[pl.ds(base + (k0 + 2) * _CH, _CH)], rb0, sr0)
                pltpu.async_copy(
                    x_hbm.at[pl.ds(base + (k0 + 2) * _CH, _CH)], xb0, sx0)
            wait_read(xb1, sx1)
            wait_read(rb1, sr1)

            @pl.when(g > 0)
            def _():
                wait_write(rb1, out_hbm, so1)
            out_buf(xb1, rb1, 1)
            pltpu.async_copy(cdf_v.at[pl.ds(_CH, _CH)],
                             out_hbm.at[pl.ds(base + (k0 + 1) * _CH, _CH)],
                             so1)
            return 0
        lax.fori_loop(0, _NC // 2, outg, 0)
        wait_write(rb0, out_hbm, so0)
        wait_write(rb1, out_hbm, so1)

    def do_ci(ci, _):
        c = wid * _CPW + ci

        def do_b(b, _):
            phase1(b, c)
            return 0
        lax.fori_loop(0, _B, do_b, 0)
        for b in range(_B):
            phase2(b, _PERM[b], c)
        return 0
    lax.fori_loop(0, _CPW, do_ci, 0)


def kernel(x):
    B, C, W, H = x.shape
    _, klmda = jax.random.split(jax.random.key(42))
    lmda = jax.random.beta(klmda, _ALPHA, _ALPHA, (B,)).astype(x.dtype)
    w_rep = jnp.broadcast_to((1.0 - lmda)[:, None], (B, 16)).reshape(-1)

    out_i32, _, _ = _mix(x.reshape(-1), w_rep)
    return lax.bitcast_convert_type(out_i32, jnp.float32).reshape(B, C, W, H)


# speedup vs baseline: 99.7075x; 1.0432x over previous
import functools

import jax
import jax.numpy as jnp
import numpy as np
from jax import lax
from jax.experimental import pallas as pl
from jax.experimental.pallas import tpu as pltpu
from jax.experimental.pallas import tpu_sc as plsc

_ALPHA = 0.1
_B, _C, _W, _H = 16, 96, 224, 224
_N = _W * _H
_NCH = _B * _C
_NBINS = 1 << 15
_NV = _N // 16
_CH = 3136
_NC = _N // _CH
_VC = _CH // 16
_CPW = _C // 32

_PERM = (1, 3, 9, 11, 5, 15, 0, 14, 2, 12, 6, 7, 13, 10, 4, 8)

_mesh = plsc.VectorSubcoreMesh(core_axis_name="c", subcore_axis_name="s")
_sc_params = pltpu.CompilerParams(needs_layout_passes=False)


def _keyaddr(v):
    bi = plsc.bitcast(v, jnp.int32)
    t = bi ^ (bi >> 31)
    lane = (plsc.bitcast(v, jnp.uint32) >> 28).astype(jnp.int32)
    return ((t >> 13) & jnp.int32(0x7FF0)) | lane


@functools.partial(
    pl.kernel,
    out_type=(
        jax.ShapeDtypeStruct((_NCH * _N,), jnp.int32),
        jax.ShapeDtypeStruct((_NCH * _N,), jnp.int32),
        jax.ShapeDtypeStruct((_NCH * _N,), jnp.float32),
    ),
    mesh=_mesh,
    compiler_params=_sc_params,
    scratch_types=[
        pltpu.VMEM((_NBINS,), jnp.int32),
        pltpu.VMEM((_N,), jnp.float32),
        pltpu.VMEM((_CH,), jnp.float32),
        pltpu.VMEM((_CH,), jnp.float32),
        pltpu.VMEM((_CH,), jnp.int32),
        pltpu.VMEM((_CH,), jnp.int32),
        pltpu.VMEM((16,), jnp.float32),
        pltpu.SemaphoreType.DMA,
        pltpu.SemaphoreType.DMA,
        pltpu.SemaphoreType.DMA,
        pltpu.SemaphoreType.DMA,
        pltpu.SemaphoreType.DMA,
        pltpu.SemaphoreType.DMA,
        pltpu.SemaphoreType.DMA,
        pltpu.SemaphoreType.DMA,
        pltpu.SemaphoreType.DMA,
        pltpu.SemaphoreType.DMA,
        pltpu.SemaphoreType.DMA,
        pltpu.SemaphoreType.DMA,
        pltpu.VMEM_SHARED((_NBINS // 8,), jnp.int32),
        pltpu.VMEM_SHARED((_N // 2,), jnp.float32),
    ],
)
def _mix(x_hbm, w_hbm, out_hbm, ranks_hbm, invcdf_hbm,
         cdf_v, inv_v, xb0, xb1, rb0, rb1, wb_v,
         sx0, sx1, sr0, sr1, sw0, sw1, so0, so1, sv, si, sz, sn,
         zero_sp, neg_sp):
    wid = lax.axis_index("c") * 16 + lax.axis_index("s")
    sid = lax.axis_index("s")
    ones = jnp.ones((16,), jnp.int32)
    zeros = jnp.zeros((16,), jnp.int32)
    neginf = jnp.full((16,), -jnp.inf, jnp.float32)
    vmaxidx = lax.iota(jnp.int32, 16) * 16 + 15
    lastlane = jnp.full((16,), 15, jnp.int32)

    def cfill(i, _):
        for j in range(4):
            rb0[pl.ds((i * 4 + j) * 16, 16)] = zeros
            xb0[pl.ds((i * 4 + j) * 16, 16)] = neginf
        return 0
    lax.fori_loop(0, _VC // 4, cfill, 0)
    pltpu.sync_copy(rb0.at[pl.ds(0, _NBINS // 128)],
                    zero_sp.at[pl.ds(sid * (_NBINS // 128), _NBINS // 128)])
    pltpu.sync_copy(xb0.at[pl.ds(0, _N // 32)],
                    neg_sp.at[pl.ds(sid * (_N // 32), _N // 32)])
    plsc.subcore_barrier()

    def wait_read(buf, sem):
        pltpu.make_async_copy(x_hbm.at[pl.ds(0, _CH)], buf, sem).wait()

    def wait_write(buf, hbm, sem):
        pltpu.make_async_copy(buf, hbm.at[pl.ds(0, _CH)], sem).wait()

    def phase1(b, c):
        ch = b * _C + c
        base = ch * _N

        @pl.when(b == 0)
        def _():
            for q in range(8):
                pltpu.sync_copy(zero_sp,
                                cdf_v.at[pl.ds(q * (_NBINS // 8), _NBINS // 8)])

        @pl.when(b > 0)
        def _():
            for q in range(8):
                pltpu.make_async_copy(
                    zero_sp, cdf_v.at[pl.ds(0, _NBINS // 8)], sz).wait()
            pltpu.make_async_copy(inv_v, invcdf_hbm.at[pl.ds(0, _N)],
                                  sv).wait()
        pltpu.async_copy(neg_sp, inv_v.at[pl.ds(0, _N // 2)], sn)
        pltpu.async_copy(neg_sp, inv_v.at[pl.ds(_N // 2, _N // 2)], sn)

        def hist_buf(xb):
            def hist7(i, _):
                for j in range(7):
                    a = _keyaddr(xb[pl.ds((i * 7 + j) * 16, 16)])
                    plsc.addupdate_scatter(cdf_v, [a], ones)
                return 0
            lax.fori_loop(0, _VC // 7, hist7, 0)

        pltpu.async_copy(x_hbm.at[pl.ds(base, _CH)], xb0, sx0)

        def histg(g, _):
            k0 = 2 * g
            pltpu.async_copy(x_hbm.at[pl.ds(base + (k0 + 1) * _CH, _CH)],
                             xb1, sx1)
            wait_read(xb0, sx0)
            hist_buf(xb0)

            @pl.when(k0 + 2 < _NC)
            def _():
                pltpu.async_copy(x_hbm.at[pl.ds(base + (k0 + 2) * _CH, _CH)],
                                 xb0, sx0)
            wait_read(xb1, sx1)
            hist_buf(xb1)
            return 0
        lax.fori_loop(0, _NC // 2, histg, 0)

        def sumA(i, acc):
            for j in range(8):
                acc = acc + cdf_v[pl.ds((i * 8 + j) * 16, 16)]
            return acc
        tot = lax.fori_loop(0, _NBINS // 128, sumA, zeros)
        ct = plsc.cumsum(tot)
        ct15 = jnp.max(ct)
        poslane = lax.iota(jnp.int32, 16) < 8
        possum = jnp.sum(jnp.where(poslane, tot, zeros))
        run0 = jnp.where(poslane, (ct15 - possum) + ct - tot, ct15 - ct)

        def scanC(i, run):
            for j in range(8):
                sl = pl.ds((i * 8 + j) * 16, 16)
                h = cdf_v[sl]
                cdf_v[sl] = run
                run = run + h
            return run
        lax.fori_loop(0, _NBINS // 128, scanC, run0)

        pltpu.make_async_copy(neg_sp, inv_v.at[pl.ds(0, _N // 2)], sn).wait()
        pltpu.make_async_copy(neg_sp, inv_v.at[pl.ds(0, _N // 2)], sn).wait()

        def rank_buf(xb, rb):
            @plsc.parallel_loop(0, _VC, 1, unroll=7)
            def rank_vreg(i):
                v = xb[pl.ds(i * 16, 16)]
                r = plsc.load_gather(cdf_v, [_keyaddr(v)])
                rb[pl.ds(i * 16, 16)] = r
                plsc.store_scatter(inv_v, [r], v)

        pltpu.async_copy(x_hbm.at[pl.ds(base, _CH)], xb0, sx0)

        def rankg(g, _):
            k0 = 2 * g
            pltpu.async_copy(x_hbm.at[pl.ds(base + (k0 + 1) * _CH, _CH)],
                             xb1, sx1)
            wait_read(xb0, sx0)

            @pl.when(g > 0)
            def _():
                wait_write(rb0, ranks_hbm, sw0)
            rank_buf(xb0, rb0)
            pltpu.async_copy(rb0, ranks_hbm.at[pl.ds(base + k0 * _CH, _CH)],
                             sw0)

            @pl.when(k0 + 2 < _NC)
            def _():
                pltpu.async_copy(x_hbm.at[pl.ds(base + (k0 + 2) * _CH, _CH)],
                                 xb0, sx0)
            wait_read(xb1, sx1)

            @pl.when(g > 0)
            def _():
                wait_write(rb1, ranks_hbm, sw1)
            rank_buf(xb1, rb1)
            pltpu.async_copy(rb1,
                             ranks_hbm.at[pl.ds(base + (k0 + 1) * _CH, _CH)],
                             sw1)
            return 0
        lax.fori_loop(0, _NC // 2, rankg, 0)
        wait_write(rb0, ranks_hbm, sw0)
        wait_write(rb1, ranks_hbm, sw1)

        @pl.when(b < _B - 1)
        def _():
            for q in range(8):
                pltpu.async_copy(
                    zero_sp, cdf_v.at[pl.ds(q * (_NBINS // 8), _NBINS // 8)],
                    sz)

        @plsc.parallel_loop(0, _NV, 1, unroll=8)
        def fillA(i):
            sl = pl.ds(i * 16, 16)
            inv_v[sl] = plsc.cummax(inv_v[sl])

        def fillB(g, carry):
            mx = plsc.load_gather(inv_v, [g * 256 + vmaxidx])
            pf = jnp.maximum(plsc.cummax(mx), carry)
            xb0[pl.ds(g * 16, 16)] = pf
            return jnp.take(pf, lastlane)
        lax.fori_loop(0, _NV // 16, fillB,
                      jnp.full((16,), -jnp.inf, jnp.float32))

        @plsc.parallel_loop(1, _NV, 1, unroll=8)
        def fillC(i):
            pv = plsc.load_gather(xb0, [zeros + (i - 1)])
            sl = pl.ds(i * 16, 16)
            inv_v[sl] = jnp.maximum(inv_v[sl], pv)

        pltpu.async_copy(inv_v, invcdf_hbm.at[pl.ds(base, _N)], sv)

    def phase2(b, pb, c):
        base = (b * _C + c) * _N
        if b == 0:
            pltpu.make_async_copy(inv_v, invcdf_hbm.at[pl.ds(0, _N)],
                                  sv).wait()
        pltpu.async_copy(invcdf_hbm.at[pl.ds((pb * _C + c) * _N, _N)],
                         inv_v, si).wait()
        pltpu.async_copy(w_hbm.at[pl.ds(b * 16, 16)], wb_v, si).wait()
        wv = wb_v[...]

        def out_buf(xb, rb, slot):
            @plsc.parallel_loop(0, _VC, 1, unroll=7)
            def out_vreg(i):
                sl = pl.ds(i * 16, 16)
                xv = xb[sl]
                m = plsc.load_gather(inv_v, [rb[sl]])
                cdf_v[pl.ds(slot * _CH + i * 16, 16)] = plsc.bitcast(
                    xv + (m - xv) * wv, jnp.int32)

        pltpu.async_copy(ranks_hbm.at[pl.ds(base, _CH)], rb0, sr0)
        pltpu.async_copy(x_hbm.at[pl.ds(base, _CH)], xb0, sx0)

        def outg(g, _):
            k0 = 2 * g
            pltpu.async_copy(ranks_hbm.at[pl.ds(base + (k0 + 1) * _CH, _CH)],
                             rb1, sr1)
            pltpu.async_copy(x_hbm.at[pl.ds(base + (k0 + 1) * _CH, _CH)],
                             xb1, sx1)
            wait_read(xb0, sx0)
            wait_read(rb0, sr0)

            @pl.when(g > 0)
            def _():
                wait_write(rb0, out_hbm, so0)
            out_buf(xb0, rb0, 0)
            pltpu.async_copy(cdf_v.at[pl.ds(0, _CH)],
                             out_hbm.at[pl.ds(base + k0 * _CH, _CH)], so0)

            @pl.when(k0 + 2 < _NC)
            def _():
                pltpu.async_copy(
                    ranks_hbm.at[pl.ds(base + (k0 + 2) * _CH, _CH)], rb0, sr0)
                pltpu.async_copy(
                    x_hbm.at[pl.ds(base + (k0 + 2) * _CH, _CH)], xb0, sx0)
            wait_read(xb1, sx1)
            wait_read(rb1, sr1)

            @pl.when(g > 0)
            def _():
                wait_write(rb1, out_hbm, so1)
            out_buf(xb1, rb1, 1)
            pltpu.async_copy(cdf_v.at[pl.ds(_CH, _CH)],
                             out_hbm.at[pl.ds(base + (k0 + 1) * _CH, _CH)],
                             so1)
            return 0
        lax.fori_loop(0, _NC // 2, outg, 0)
        wait_write(rb0, out_hbm, so0)
        wait_write(rb1, out_hbm, so1)

    def do_ci(ci, _):
        c = wid * _CPW + ci

        def do_b(b, _):
            phase1(b, c)
            return 0
        lax.fori_loop(0, _B, do_b, 0)
        for b in range(_B):
            phase2(b, _PERM[b], c)
        return 0
    lax.fori_loop(0, _CPW, do_ci, 0)


def kernel(x):
    B, C, W, H = x.shape
    _, klmda = jax.random.split(jax.random.key(42))
    lmda = jax.random.beta(klmda, _ALPHA, _ALPHA, (B,)).astype(x.dtype)
    w_rep = jnp.broadcast_to((1.0 - lmda)[:, None], (B, 16)).reshape(-1)

    out_i32, _, _ = _mix(x.reshape(-1), w_rep)
    return lax.bitcast_convert_type(out_i32, jnp.float32).reshape(B, C, W, H)
